# Initial kernel scaffold; baseline (speedup 1.0000x reference)
import jax
import jax.numpy as jnp
from jax.experimental import pallas as pl


def kernel(idx, targets, table):
    raise NotImplementedError("write your pallas kernel here")


# speedup vs baseline: 1.6116x; 1.6116x over previous
import functools

import jax
import jax.numpy as jnp
from jax import lax
from jax.experimental import pallas as pl
from jax.experimental.pallas import tpu as pltpu
from jax.experimental.pallas import tpu_sc as plsc

_V = 1000
_N = 51200
_LANES = 16

_NW = 32
_PER_W = _N // _NW
_CHUNK = 32
_STEPS = _PER_W // _CHUNK


def _lse_body(table_ref, lse_ref):
    x = table_ref[...]
    m = jnp.max(x, axis=1)
    s = jnp.sum(jnp.exp(x - m[:, None]), axis=1)
    lse_ref[...] = jnp.pad(jnp.log(s) + m, (0, _LANES + 8))


def _table_lse(table):
    return pl.pallas_call(
        _lse_body,
        out_shape=jax.ShapeDtypeStruct((_V + _LANES + 8,), jnp.float32),
    )(table)


def _sc_body(table_hbm, idx_hbm, tgt_hbm, lse_hbm,
             out_hbm, psum_hbm,
             idx_v, tgt_v, lse_v, rows_v, acc_v, sem):
    wid = lax.axis_index("s") * 2 + lax.axis_index("c")
    base = wid * _PER_W

    pltpu.sync_copy(idx_hbm.at[pl.ds(base, _PER_W)], idx_v)
    pltpu.sync_copy(tgt_hbm.at[pl.ds(base, _PER_W)], tgt_v)
    pltpu.sync_copy(lse_hbm, lse_v)

    def step(c, acc):
        o = c * _CHUNK
        pltpu.async_copy(
            table_hbm.at[idx_v.at[pl.ds(o, _CHUNK)]],
            rows_v.at[pl.ds(0, _CHUNK)], sem).wait()
        pltpu.sync_copy(
            rows_v.at[pl.ds(0, _CHUNK)], out_hbm.at[pl.ds(base + o, _CHUNK)])

        for j in range(_CHUNK // _LANES):
            iv16 = idx_v[pl.ds(o + j * _LANES, _LANES)]
            tv16 = tgt_v[pl.ds(o + j * _LANES, _LANES)]
            for k in range(_LANES):
                i = j * _LANES + k
                lse16 = lse_v[pl.ds(iv16[k], _LANES)]
                row16 = rows_v[i, pl.ds(tv16[k], _LANES)]
                acc = acc + (lse16 - row16)
        return acc

    zero = jnp.zeros((_LANES,), jnp.float32)
    acc = lax.fori_loop(0, _STEPS, step, zero)
    lane = lax.broadcasted_iota(jnp.int32, (_LANES,), 0)
    acc_v[...] = jnp.where(lane == 0, acc, zero)
    pltpu.sync_copy(acc_v, psum_hbm.at[wid])


@functools.partial(
    pl.kernel,
    mesh=plsc.VectorSubcoreMesh(core_axis_name="c", subcore_axis_name="s"),
    compiler_params=pltpu.CompilerParams(use_tc_tiling_on_sc=False),
    out_type=[
        jax.ShapeDtypeStruct((_N, _V), jnp.float32),
        jax.ShapeDtypeStruct((_NW, _LANES), jnp.float32),
    ],
    scratch_types=[
        pltpu.VMEM((_PER_W,), jnp.int32),
        pltpu.VMEM((_PER_W,), jnp.int32),
        pltpu.VMEM((_V + _LANES + 8,), jnp.float32),
        pltpu.VMEM((_CHUNK + 1, _V), jnp.float32),
        pltpu.VMEM((_LANES,), jnp.float32),
        pltpu.SemaphoreType.DMA,
    ],
)
def _sc_gather(*args):
    _sc_body(*args)


def kernel(idx, targets, table):
    idx_f = idx.reshape(_N).astype(jnp.int32)
    tgt_f = targets.reshape(_N).astype(jnp.int32)
    lse = _table_lse(table)
    logits, psum = _sc_gather(table, idx_f, tgt_f, lse)
    loss = jnp.sum(psum) * (1.0 / _N)
    return logits, loss


# speedup vs baseline: 1.7117x; 1.0621x over previous
import functools

import jax
import jax.numpy as jnp
from jax import lax
from jax.experimental import pallas as pl
from jax.experimental.pallas import tpu as pltpu
from jax.experimental.pallas import tpu_sc as plsc

_V = 1000
_N = 51200
_LANES = 16

_NW = 32
_PER_W = _N // _NW
_CHUNK = 32
_STEPS = _PER_W // _CHUNK


def _lse_body(table_ref, lse_ref):
    x = table_ref[...]
    m = jnp.max(x, axis=1)
    s = jnp.sum(jnp.exp(x - m[:, None]), axis=1)
    lse_ref[...] = jnp.pad(jnp.log(s) + m, (0, _LANES + 8))


def _table_lse(table):
    return pl.pallas_call(
        _lse_body,
        out_shape=jax.ShapeDtypeStruct((_V + _LANES + 8,), jnp.float32),
    )(table)


def _sc_body(table_hbm, idx_hbm, tgt_hbm, lse_hbm,
             out_hbm, psum_hbm,
             idx_v, tgt_v, lse_v, rows0_v, rows1_v, acc_v,
             sg0, sg1, sw0, sw1):
    wid = lax.axis_index("s") * 2 + lax.axis_index("c")
    base = wid * _PER_W

    pltpu.sync_copy(idx_hbm.at[pl.ds(base, _PER_W)], idx_v)
    pltpu.sync_copy(tgt_hbm.at[pl.ds(base, _PER_W)], tgt_v)
    pltpu.sync_copy(lse_hbm, lse_v)

    rows = (rows0_v, rows1_v)
    sg = (sg0, sg1)
    sw = (sw0, sw1)

    def gather_start(c, b):
        pltpu.async_copy(
            table_hbm.at[idx_v.at[pl.ds(c * _CHUNK, _CHUNK)]],
            rows[b].at[pl.ds(0, _CHUNK)], sg[b])

    def gather_wait(c, b):
        pltpu.make_async_copy(
            table_hbm.at[idx_v.at[pl.ds(c * _CHUNK, _CHUNK)]],
            rows[b].at[pl.ds(0, _CHUNK)], sg[b]).wait()

    def write_start(c, b):
        pltpu.async_copy(
            rows[b].at[pl.ds(0, _CHUNK)],
            out_hbm.at[pl.ds(base + c * _CHUNK, _CHUNK)], sw[b])

    def write_wait(c, b):
        pltpu.make_async_copy(
            rows[b].at[pl.ds(0, _CHUNK)],
            out_hbm.at[pl.ds(base + c * _CHUNK, _CHUNK)], sw[b]).wait()

    def loss_chunk(c, b, acc):
        o = c * _CHUNK
        for j in range(_CHUNK // _LANES):
            iv16 = idx_v[pl.ds(o + j * _LANES, _LANES)]
            tv16 = tgt_v[pl.ds(o + j * _LANES, _LANES)]
            for k in range(_LANES):
                i = j * _LANES + k
                lse16 = lse_v[pl.ds(iv16[k], _LANES)]
                row16 = rows[b][i, pl.ds(tv16[k], _LANES)]
                acc = acc + (lse16 - row16)
        return acc

    gather_start(0, 0)

    def outer(o, acc):
        for b in range(2):
            c = 2 * o + b
            if b == 0:
                @pl.when(o >= 1)
                def _():
                    write_wait(c - 1, 1)
                gather_start(c + 1, 1)
            else:
                write_wait(c - 1, 0)

                @pl.when(o < _STEPS // 2 - 1)
                def _():
                    gather_start(c + 1, 0)
            gather_wait(c, b)
            write_start(c, b)
            acc = loss_chunk(c, b, acc)
        return acc

    zero = jnp.zeros((_LANES,), jnp.float32)
    acc = lax.fori_loop(0, _STEPS // 2, outer, zero)
    write_wait(_STEPS - 1, 1)
    lane = lax.broadcasted_iota(jnp.int32, (_LANES,), 0)
    acc_v[...] = jnp.where(lane == 0, acc, zero)
    pltpu.sync_copy(acc_v, psum_hbm.at[wid])


@functools.partial(
    pl.kernel,
    mesh=plsc.VectorSubcoreMesh(core_axis_name="c", subcore_axis_name="s"),
    compiler_params=pltpu.CompilerParams(use_tc_tiling_on_sc=False),
    out_type=[
        jax.ShapeDtypeStruct((_N, _V), jnp.float32),
        jax.ShapeDtypeStruct((_NW, _LANES), jnp.float32),
    ],
    scratch_types=[
        pltpu.VMEM((_PER_W,), jnp.int32),
        pltpu.VMEM((_PER_W,), jnp.int32),
        pltpu.VMEM((_V + _LANES + 8,), jnp.float32),
        pltpu.VMEM((_CHUNK + 1, _V), jnp.float32),
        pltpu.VMEM((_CHUNK + 1, _V), jnp.float32),
        pltpu.VMEM((_LANES,), jnp.float32),
        pltpu.SemaphoreType.DMA,
        pltpu.SemaphoreType.DMA,
        pltpu.SemaphoreType.DMA,
        pltpu.SemaphoreType.DMA,
    ],
)
def _sc_gather(*args):
    _sc_body(*args)


def kernel(idx, targets, table):
    idx_f = idx.reshape(_N).astype(jnp.int32)
    tgt_f = targets.reshape(_N).astype(jnp.int32)
    lse = _table_lse(table)
    logits, psum = _sc_gather(table, idx_f, tgt_f, lse)
    loss = jnp.sum(psum) * (1.0 / _N)
    return logits, loss


# speedup vs baseline: 2.7993x; 1.6354x over previous
import functools

import jax
import jax.numpy as jnp
from jax import lax
from jax.experimental import pallas as pl
from jax.experimental.pallas import tpu as pltpu
from jax.experimental.pallas import tpu_sc as plsc

_V = 1000
_VP = 1024
_N = 51200
_LANES = 16

_NW = 32
_PER_W = _N // _NW
_CHUNK = 32
_STEPS = _PER_W // _CHUNK
_TLC = 80


def _lse_body(table_ref, lse_ref):
    x = table_ref[...]
    m = jnp.max(x, axis=1)
    s = jnp.sum(jnp.exp(x - m[:, None]), axis=1)
    lse_ref[...] = jnp.pad(jnp.log(s) + m, (0, _VP - _V))


def _table_lse(table):
    return pl.pallas_call(
        _lse_body,
        out_shape=jax.ShapeDtypeStruct((_VP,), jnp.float32),
    )(table)


def _sc_body(table_hbm, flat_hbm, idx_hbm, tgt_hbm, lse_hbm,
             out_hbm, psum_hbm,
             idx_v, fi_v, lse_v, tl_v, rows0_v, rows1_v, acc_v,
             sg0, sg1, sw0, sw1, st):
    wid = lax.axis_index("s") * 2 + lax.axis_index("c")
    base = wid * _PER_W

    pltpu.sync_copy(idx_hbm.at[pl.ds(base, _PER_W)], idx_v)
    pltpu.sync_copy(tgt_hbm.at[pl.ds(base, _PER_W)], fi_v)
    pltpu.sync_copy(lse_hbm, lse_v)

    def flatten(i, _):
        o = i * _LANES
        fi_v[pl.ds(o, _LANES)] = (
            idx_v[pl.ds(o, _LANES)] * _V + fi_v[pl.ds(o, _LANES)])
        return 0

    lax.fori_loop(0, _PER_W // _LANES, flatten, 0)

    def tl_start(g, _):
        pltpu.async_copy(
            flat_hbm.at[fi_v.at[pl.ds(g * _TLC, _TLC)]],
            tl_v.at[pl.ds(g * _TLC, _TLC)], st)
        return 0

    lax.fori_loop(0, _PER_W // _TLC, tl_start, 0)

    rows = (rows0_v, rows1_v)
    sg = (sg0, sg1)
    sw = (sw0, sw1)

    def gather_start(c, b):
        pltpu.async_copy(
            table_hbm.at[idx_v.at[pl.ds(c * _CHUNK, _CHUNK)]],
            rows[b], sg[b])

    def gather_wait(c, b):
        pltpu.make_async_copy(
            table_hbm.at[idx_v.at[pl.ds(c * _CHUNK, _CHUNK)]],
            rows[b], sg[b]).wait()

    def write_start(c, b):
        pltpu.async_copy(
            rows[b], out_hbm.at[pl.ds(base + c * _CHUNK, _CHUNK)], sw[b])

    def write_wait(c, b):
        pltpu.make_async_copy(
            rows[b], out_hbm.at[pl.ds(base + c * _CHUNK, _CHUNK)], sw[b]).wait()

    gather_start(0, 0)

    def outer(o, _):
        for b in range(2):
            c = 2 * o + b
            if b == 0:
                @pl.when(o >= 1)
                def _w():
                    write_wait(c - 1, 1)
                gather_start(c + 1, 1)
            else:
                write_wait(c - 1, 0)

                @pl.when(o < _STEPS // 2 - 1)
                def _w():
                    gather_start(c + 1, 0)
            gather_wait(c, b)
            write_start(c, b)
        return 0

    lax.fori_loop(0, _STEPS // 2, outer, 0)

    def tl_drain(g, _):
        pltpu.make_async_copy(
            flat_hbm.at[fi_v.at[pl.ds(g * _TLC, _TLC)]],
            tl_v.at[pl.ds(g * _TLC, _TLC)], st).wait()
        return 0

    lax.fori_loop(0, _PER_W // _TLC, tl_drain, 0)

    def loss_grp(j, acc):
        o = j * _LANES
        iv16 = idx_v[pl.ds(o, _LANES)]
        for k in range(_LANES):
            lse16 = lse_v[pl.ds(iv16[k], _LANES)]
            tl16 = tl_v[pl.ds(o + k, _LANES)]
            acc = acc + (lse16 - tl16)
        return acc

    zero = jnp.zeros((_LANES,), jnp.float32)
    acc = lax.fori_loop(0, _PER_W // _LANES, loss_grp, zero)
    write_wait(_STEPS - 1, 1)
    lane = lax.broadcasted_iota(jnp.int32, (_LANES,), 0)
    acc_v[...] = jnp.where(lane == 0, acc, zero)
    pltpu.sync_copy(acc_v, psum_hbm.at[pl.ds(wid * _LANES, _LANES)])


@functools.partial(
    pl.kernel,
    mesh=plsc.VectorSubcoreMesh(core_axis_name="c", subcore_axis_name="s"),
    out_type=[
        jax.ShapeDtypeStruct((_N, _VP), jnp.float32),
        jax.ShapeDtypeStruct((_NW * _LANES,), jnp.float32),
    ],
    scratch_types=[
        pltpu.VMEM((_PER_W,), jnp.int32),
        pltpu.VMEM((_PER_W,), jnp.int32),
        pltpu.VMEM((_VP,), jnp.float32),
        pltpu.VMEM((_PER_W + _LANES,), jnp.float32),
        pltpu.VMEM((_CHUNK, _VP), jnp.float32),
        pltpu.VMEM((_CHUNK, _VP), jnp.float32),
        pltpu.VMEM((_LANES,), jnp.float32),
        pltpu.SemaphoreType.DMA,
        pltpu.SemaphoreType.DMA,
        pltpu.SemaphoreType.DMA,
        pltpu.SemaphoreType.DMA,
        pltpu.SemaphoreType.DMA,
    ],
)
def _sc_gather(*args):
    _sc_body(*args)


def kernel(idx, targets, table):
    idx_f = idx.reshape(_N).astype(jnp.int32)
    tgt_f = targets.reshape(_N).astype(jnp.int32)
    lse = _table_lse(table)
    table_pad = jnp.pad(table, ((0, 0), (0, _VP - _V)))
    flat = table.reshape(_V * _V)
    logits_pad, psum = _sc_gather(table_pad, flat, idx_f, tgt_f, lse)
    loss = jnp.sum(psum) * (1.0 / _N)
    return logits_pad[:, :_V], loss
